# baseline (speedup 1.0000x reference)
import jax
import jax.numpy as jnp
from jax import lax
from jax.experimental import pallas as pl
from jax.experimental.pallas import tpu as pltpu
from jax.experimental.pallas import tpu_sc as plsc

N = 10000
F = 128
D = 128
DH = D // 2
KHOP = 3
E = 320000
NC, NS, L = 2, 16, 16
NW = NC * NS
CHUNK = 128
EPWP = 10240
EPH = EPWP * NW
NCHP = EPWP // CHUNK
NP = 10240
NPP = NP // 2
RPT = NPP // NS

BN = 1000


def _dense_body(x_ref, w1_ref, b1_ref, w2_ref, b2_ref, wb_ref, wf_ref,
                wr_ref, a_ref, mn_ref, dense_ref):
    x = x_ref[...]
    h = jnp.maximum(
        jnp.dot(x, w1_ref[...], preferred_element_type=jnp.float32)
        + b1_ref[...][None, :], 0.0)
    le = (jnp.dot(h, w2_ref[...], preferred_element_type=jnp.float32)
          + b2_ref[...][None, :])
    a = jnp.clip(a_ref[...], 0.0, 1.0)
    wcomb = jnp.sum(a) * wb_ref[...] + jnp.sum(
        a[:, None, None] * wr_ref[...], axis=0)
    dense_ref[...] = jnp.dot(le, wcomb, preferred_element_type=jnp.float32)
    for hop in range(KHOP):
        mn_ref[hop] = jnp.dot(x, wf_ref[hop],
                              preferred_element_type=jnp.float32)


def _dense_call(x, w1, b1, w2, b2, wb, wf, wr, a):
    full = lambda shape: pl.BlockSpec(shape, lambda i: tuple(0 for _ in shape))
    return pl.pallas_call(
        _dense_body,
        grid=(N // BN,),
        in_specs=[
            pl.BlockSpec((BN, F), lambda i: (i, 0)),
            full((F, 2 * D)),
            full((2 * D,)),
            full((2 * D, D)),
            full((D,)),
            full((D, D)),
            full((KHOP, F, D)),
            full((KHOP, D, D)),
            full((KHOP,)),
        ],
        out_specs=[
            pl.BlockSpec((KHOP, BN, D), lambda i: (0, i, 0)),
            pl.BlockSpec((BN, D), lambda i: (i, 0)),
        ],
        out_shape=[
            jax.ShapeDtypeStruct((KHOP, N, D), jnp.float32),
            jax.ShapeDtypeStruct((N, D), jnp.float32),
        ],
    )(x, w1, b1, w2, b2, wb, wf, wr, a)


def _sc_body(mn2_hbm, srcp_hbm, dstp_hbm, valp_hbm, zeros_hbm, out_hbm,
             acc_sh, mn_sh, src_v, dst_v, val_v, gidx_v, sidx_v,
             rows_g, rows_s, sem):
    c = lax.axis_index("c")
    s = lax.axis_index("s")
    w = s * NC + c

    def chunk_body(i, carry, h):
        base = h * EPH + w * EPWP + i * CHUNK
        pltpu.sync_copy(srcp_hbm.at[pl.ds(base, CHUNK)], src_v)
        pltpu.sync_copy(dstp_hbm.at[pl.ds(base, CHUNK)], dst_v)
        pltpu.sync_copy(valp_hbm.at[pl.ds(base, CHUNK)], val_v)

        def conv_group(g, carry2):
            sl = pl.ds(g * L, L)
            gidx_v[sl] = jax.lax.shift_right_logical(src_v[sl], 1)
            sidx_v[sl] = jax.lax.shift_right_logical(dst_v[sl], 1)
            return carry2

        lax.fori_loop(0, CHUNK // L, conv_group, 0)
        pltpu.async_copy(mn_sh.at[gidx_v], rows_g, sem).wait()

        def scale_group(g, carry2):
            sl = pl.ds(g * L, L)
            grp_src = src_v[sl]
            grp_dst = dst_v[sl]
            grp_val = val_v[sl]
            dpf = (grp_dst & 1).astype(jnp.float32)
            mh = grp_val * dpf
            ml = grp_val - mh
            for lane in range(L):
                e = g * L + lane
                off = (grp_src[lane] & 1) * DH
                a = ml[lane]
                b = mh[lane]
                for q in range(DH // L):
                    sel = rows_g[e, pl.ds(off + q * L, L)]
                    rows_s[e, pl.ds(q * L, L)] = sel * a
                    rows_s[e, pl.ds(DH + q * L, L)] = sel * b
            return carry2

        lax.fori_loop(0, CHUNK // L, scale_group, 0)
        pltpu.sync_copy(rows_s, acc_sh.at[sidx_v], add=True)
        return carry

    rsl = pl.ds(s * RPT, RPT)
    for half in range(2):
        pltpu.sync_copy(zeros_hbm.at[rsl], acc_sh.at[rsl])
        plsc.subcore_barrier()
        for h in range(KHOP):
            pltpu.sync_copy(mn2_hbm.at[h, half, rsl], mn_sh.at[rsl])
            plsc.subcore_barrier()
            lax.fori_loop(0, NCHP, lambda i, cc, h=h: chunk_body(i, cc, h), 0)
            plsc.subcore_barrier()
        pltpu.sync_copy(acc_sh.at[rsl],
                        out_hbm.at[half, pl.ds(c * NPP + s * RPT, RPT)])
        plsc.subcore_barrier()


_sc_call = pl.kernel(
    _sc_body,
    out_type=jax.ShapeDtypeStruct((2, NC * NPP, D), jnp.float32),
    mesh=plsc.VectorSubcoreMesh(core_axis_name="c", subcore_axis_name="s"),
    scratch_types=[
        pltpu.VMEM_SHARED((NPP, D), jnp.float32),
        pltpu.VMEM_SHARED((NPP, D), jnp.float32),
        pltpu.VMEM((CHUNK,), jnp.int32),
        pltpu.VMEM((CHUNK,), jnp.int32),
        pltpu.VMEM((CHUNK,), jnp.float32),
        pltpu.VMEM((CHUNK,), jnp.int32),
        pltpu.VMEM((CHUNK,), jnp.int32),
        pltpu.VMEM((CHUNK, D), jnp.float32),
        pltpu.VMEM((CHUNK, D), jnp.float32),
        pltpu.SemaphoreType.DMA,
    ],
)


def _finish_body(acc_ref, dense_ref, out_ref):
    lo = acc_ref[0, 0] + acc_ref[0, 1]
    hi = acc_ref[1, 0] + acc_ref[1, 1]
    out_ref[...] = jnp.maximum(
        jnp.concatenate([lo, hi], axis=1) + dense_ref[...], 0.0)


def _finish_call(accs, dense):
    return pl.pallas_call(
        _finish_body,
        grid=(N // BN,),
        in_specs=[
            pl.BlockSpec((2, NC, BN, DH), lambda i: (0, 0, i, 0)),
            pl.BlockSpec((BN, D), lambda i: (i, 0)),
        ],
        out_specs=pl.BlockSpec((BN, D), lambda i: (i, 0)),
        out_shape=jax.ShapeDtypeStruct((N, D), jnp.float32),
    )(accs, dense)


def kernel(node_features, edge_index, adj_values, W_emb1, b_emb1, W_emb2,
           b_emb2, W_base, W_feat, W_res, alpha):
    mn, dense = _dense_call(node_features, W_emb1, b_emb1, W_emb2, b_emb2,
                            W_base, W_feat, W_res, alpha)
    mnp = jnp.pad(mn, ((0, 0), (0, NP - N), (0, 0)))
    mn2 = (mnp.reshape(KHOP, NPP, 2, 2, DH)
           .transpose(0, 3, 1, 2, 4)
           .reshape(KHOP, 2, NPP, D))

    src = edge_index[:, 0, :]
    dst = edge_index[:, 1, :]
    pad = EPH - E
    srcp = jnp.pad(src, ((0, 0), (0, pad))).reshape(-1)
    dstp = jnp.pad(dst, ((0, 0), (0, pad))).reshape(-1)
    valp = jnp.pad(adj_values, ((0, 0), (0, pad))).reshape(-1)
    zeros = jnp.zeros((NPP, D), jnp.float32)

    accs = _sc_call(mn2, srcp, dstp, valp, zeros)
    accs = accs.reshape(2, NC, NP, DH)[:, :, :N, :]
    return _finish_call(accs, dense)


# speedup vs baseline: 3.3365x; 3.3365x over previous
import functools

import jax
import jax.numpy as jnp
from jax import lax
from jax.experimental import pallas as pl
from jax.experimental.pallas import tpu as pltpu
from jax.experimental.pallas import tpu_sc as plsc

N = 10000
F = 128
D = 128
KHOP = 3
E = 320000
NC, NS, L = 2, 16, 16
NW = NC * NS
ET = KHOP * E
CROWS = 2
CHUNK = 320
ETP = ((ET + NW * CHUNK - 1) // (NW * CHUNK)) * NW * CHUNK
EPW = ETP // NW
NCHUNK = EPW // CHUNK
NP = 10240
ROWS_PER_TILE = NP // NS

BN = 1000


def _dense_body(x_ref, w1_ref, b1_ref, w2_ref, b2_ref, wb_ref, wf_ref,
                wr_ref, a_ref, mn_ref, dense_ref):
    x = x_ref[...]
    h = jnp.maximum(
        jnp.dot(x, w1_ref[...], preferred_element_type=jnp.float32)
        + b1_ref[...][None, :], 0.0)
    le = (jnp.dot(h, w2_ref[...], preferred_element_type=jnp.float32)
          + b2_ref[...][None, :])
    a = jnp.clip(a_ref[...], 0.0, 1.0)
    wcomb = jnp.sum(a) * wb_ref[...] + jnp.sum(
        a[:, None, None] * wr_ref[...], axis=0)
    dense_ref[...] = jnp.dot(le, wcomb, preferred_element_type=jnp.float32)
    for hop in range(KHOP):
        mn_ref[hop] = jnp.dot(x, wf_ref[hop],
                              preferred_element_type=jnp.float32)


def _dense_call(x, w1, b1, w2, b2, wb, wf, wr, a):
    grid = (N // BN,)
    full = lambda shape: pl.BlockSpec(shape, lambda i: tuple(0 for _ in shape))
    return pl.pallas_call(
        _dense_body,
        grid=grid,
        in_specs=[
            pl.BlockSpec((BN, F), lambda i: (i, 0)),
            full((F, 2 * D)),
            full((2 * D,)),
            full((2 * D, D)),
            full((D,)),
            full((D, D)),
            full((KHOP, F, D)),
            full((KHOP, D, D)),
            full((KHOP,)),
        ],
        out_specs=[
            pl.BlockSpec((KHOP, BN, D), lambda i: (0, i, 0)),
            pl.BlockSpec((BN, D), lambda i: (i, 0)),
        ],
        out_shape=[
            jax.ShapeDtypeStruct((KHOP, N, D), jnp.float32),
            jax.ShapeDtypeStruct((N, D), jnp.float32),
        ],
    )(x, w1, b1, w2, b2, wb, wf, wr, a)


def _sc_body(mn_hbm, srcg_hbm, dst_hbm, val_hbm, zeros_hbm, out_hbm,
             acc_sh, idx_v, dst_v, val_v, rows_v, sem):
    c = lax.axis_index("c")
    s = lax.axis_index("s")
    w = s * NC + c

    pltpu.sync_copy(zeros_hbm.at[pl.ds(s * ROWS_PER_TILE, ROWS_PER_TILE)],
                    acc_sh.at[pl.ds(s * ROWS_PER_TILE, ROWS_PER_TILE)])
    plsc.subcore_barrier()

    def chunk_body(i, carry):
        base = w * EPW + i * CHUNK
        pltpu.sync_copy(srcg_hbm.at[pl.ds(base, CHUNK)], idx_v)
        pltpu.sync_copy(dst_hbm.at[pl.ds(base, CHUNK)], dst_v)
        pltpu.sync_copy(val_hbm.at[pl.ds(base, CHUNK)], val_v)
        pltpu.async_copy(mn_hbm.at[idx_v], rows_v, sem).wait()

        def scale_group(g, carry2):
            grp = val_v[pl.ds(g * L, L)]
            for lane in range(L):
                v = grp[lane]
                e = g * L + lane
                for j in range(D // L):
                    sl = pl.ds(j * L, L)
                    rows_v[e, sl] = rows_v[e, sl] * v
            return carry2

        lax.fori_loop(0, CHUNK // L, scale_group, 0)
        pltpu.sync_copy(rows_v, acc_sh.at[dst_v], add=True)
        return carry

    lax.fori_loop(0, NCHUNK, chunk_body, 0)
    plsc.subcore_barrier()

    pltpu.sync_copy(
        acc_sh.at[pl.ds(s * ROWS_PER_TILE, ROWS_PER_TILE)],
        out_hbm.at[pl.ds(c * NP + s * ROWS_PER_TILE, ROWS_PER_TILE)])


_sc_call = pl.kernel(
    _sc_body,
    out_type=jax.ShapeDtypeStruct((NC * NP, D), jnp.float32),
    mesh=plsc.VectorSubcoreMesh(core_axis_name="c", subcore_axis_name="s"),
    scratch_types=[
        pltpu.VMEM_SHARED((NP, D), jnp.float32),
        pltpu.VMEM((CHUNK,), jnp.int32),
        pltpu.VMEM((CHUNK,), jnp.int32),
        pltpu.VMEM((CHUNK,), jnp.float32),
        pltpu.VMEM((CHUNK, D), jnp.float32),
        pltpu.SemaphoreType.DMA,
    ],
)


def _finish_body(acc_ref, dense_ref, out_ref):
    out_ref[...] = jnp.maximum(acc_ref[0] + acc_ref[1] + dense_ref[...], 0.0)


def _finish_call(accs, dense):
    return pl.pallas_call(
        _finish_body,
        grid=(N // BN,),
        in_specs=[
            pl.BlockSpec((NC, BN, D), lambda i: (0, i, 0)),
            pl.BlockSpec((BN, D), lambda i: (i, 0)),
        ],
        out_specs=pl.BlockSpec((BN, D), lambda i: (i, 0)),
        out_shape=jax.ShapeDtypeStruct((N, D), jnp.float32),
    )(accs, dense)


def kernel(node_features, edge_index, adj_values, W_emb1, b_emb1, W_emb2,
           b_emb2, W_base, W_feat, W_res, alpha):
    mn, dense = _dense_call(node_features, W_emb1, b_emb1, W_emb2, b_emb2,
                            W_base, W_feat, W_res, alpha)
    mn_flat = mn.reshape(KHOP * N, D)

    src = edge_index[:, 0, :]
    dst = edge_index[:, 1, :]
    srcg = (src + (jnp.arange(KHOP, dtype=jnp.int32) * N)[:, None]).reshape(-1)
    dstf = dst.reshape(-1)
    valf = adj_values.reshape(-1)
    pad = ETP - ET
    srcg = jnp.concatenate([srcg, jnp.zeros((pad,), jnp.int32)])
    dstf = jnp.concatenate([dstf, jnp.zeros((pad,), jnp.int32)])
    valf = jnp.concatenate([valf, jnp.zeros((pad,), jnp.float32)])
    zeros = jnp.zeros((NP, D), jnp.float32)

    accs = _sc_call(mn_flat, srcg, dstf, valf, zeros)
    accs = accs.reshape(NC, NP, D)[:, :N, :]
    return _finish_call(accs, dense)
